# baseline (speedup 1.0000x reference)
import functools

import jax
import jax.numpy as jnp
from jax import lax
from jax.experimental import pallas as pl
from jax.experimental.pallas import tpu as pltpu
from jax.experimental.pallas import tpu_sc as plsc

NC = 2
NS = 16
NW = NC * NS
L = 16
CHUNK = 128
PC = 16


def _gcn_sc_kernel(n_pad, d_feat, n_chunks):
    mesh = plsc.VectorSubcoreMesh(core_axis_name="c", subcore_axis_name="s")
    rows_per_tile = n_pad // NS

    @functools.partial(
        pl.kernel,
        out_type=jax.ShapeDtypeStruct((NC, n_pad, d_feat), jnp.float32),
        mesh=mesh,
        scratch_types=[
            pltpu.VMEM((PC, CHUNK), jnp.int32),
            pltpu.VMEM((PC, CHUNK), jnp.int32),
            pltpu.VMEM((PC, CHUNK), jnp.float32),
            pltpu.VMEM((CHUNK, d_feat), jnp.float32),
            pltpu.VMEM((CHUNK, d_feat), jnp.float32),
            pltpu.VMEM_SHARED((n_pad, d_feat), jnp.float32),
            pltpu.SemaphoreType.DMA,
            pltpu.SemaphoreType.DMA,
            pltpu.SemaphoreType.DMA,
            pltpu.SemaphoreType.DMA,
        ],
    )
    def k(dst_hbm, src_hbm, val_hbm, emb_hbm, out_hbm,
          dst_v, src_v, val_v, rows0, rows1, acc_sh, gs0, gs1, ss0, ss1):
        cid = lax.axis_index("c")
        sid = lax.axis_index("s")
        w = cid * NS + sid
        my_rows = pl.ds(sid * rows_per_tile, rows_per_tile)
        n_pass = n_chunks // PC

        zv = jnp.zeros((L,), jnp.float32)

        @pl.loop(0, CHUNK)
        def _(r):
            for j in range(d_feat // L):
                rows0[r, pl.ds(j * L, L)] = zv

        @pl.loop(0, rows_per_tile, step=CHUNK)
        def _(r0):
            pltpu.sync_copy(
                rows0, acc_sh.at[pl.ds(sid * rows_per_tile + r0, CHUNK)])

        plsc.subcore_barrier()

        def scale(buf, c):
            @pl.loop(0, CHUNK, step=L)
            def _(e0):
                vv = val_v[c, pl.ds(e0, L)]
                for i in range(L):
                    v = vv[i]
                    for j in range(d_feat // L):
                        sl = pl.ds(j * L, L)
                        buf[e0 + i, sl] = buf[e0 + i, sl] * v

        @pl.loop(0, n_pass)
        def _(h):
            hs = pl.ds(h * PC, PC)
            pltpu.sync_copy(dst_hbm.at[w, hs], dst_v)
            pltpu.sync_copy(src_hbm.at[w, hs], src_v)
            pltpu.sync_copy(val_hbm.at[w, hs], val_v)

            pltpu.async_copy(emb_hbm.at[src_v.at[0]], rows0, gs0)
            pltpu.async_copy(emb_hbm.at[src_v.at[1]], rows1, gs1)

            @pl.loop(0, PC, step=2)
            def _(c):
                pltpu.make_async_copy(emb_hbm.at[src_v.at[c]], rows0, gs0).wait()
                scale(rows0, c)
                pltpu.async_copy(rows0, acc_sh.at[dst_v.at[c]], ss0, add=True)

                pltpu.make_async_copy(emb_hbm.at[src_v.at[c + 1]], rows1, gs1).wait()
                scale(rows1, c + 1)
                pltpu.async_copy(rows1, acc_sh.at[dst_v.at[c + 1]], ss1, add=True)

                pltpu.make_async_copy(rows0, acc_sh.at[dst_v.at[c]], ss0).wait()

                @pl.when(c + 2 < PC)
                def _():
                    pltpu.async_copy(emb_hbm.at[src_v.at[c + 2]], rows0, gs0)

                pltpu.make_async_copy(rows1, acc_sh.at[dst_v.at[c + 1]], ss1).wait()

                @pl.when(c + 3 < PC)
                def _():
                    pltpu.async_copy(emb_hbm.at[src_v.at[c + 3]], rows1, gs1)

        plsc.subcore_barrier()
        pltpu.sync_copy(acc_sh.at[my_rows], out_hbm.at[cid, my_rows])

    return k


def _partial_add(partials, n_nodes):
    nc, n, d = partials.shape
    blk = 8 * NW
    assert n % blk == 0

    def body(p_ref, o_ref):
        o_ref[...] = p_ref[0] + p_ref[1]

    return pl.pallas_call(
        body,
        out_shape=jax.ShapeDtypeStruct((n_nodes, d), jnp.float32),
        grid=(n // blk,),
        in_specs=[pl.BlockSpec((nc, blk, d), lambda i: (0, i, 0))],
        out_specs=pl.BlockSpec((blk, d), lambda i: (i, 0)),
    )(partials)


def kernel(edge_index, edge_vals, embeds):
    n_nodes, d_feat = embeds.shape
    n_edges = edge_vals.shape[0]
    n_chunks = -(--(-n_edges // (NW * CHUNK)) // PC) * PC
    per_tile = n_chunks * CHUNK
    pad = NW * per_tile - n_edges

    dst = edge_index[0].astype(jnp.int32)
    src = edge_index[1].astype(jnp.int32)
    vals = edge_vals.astype(jnp.float32)
    ipad = jnp.arange(pad, dtype=jnp.int32) % jnp.int32(n_nodes)
    dst3 = jnp.concatenate([dst, ipad]).reshape(NW, n_chunks, CHUNK)
    src3 = jnp.concatenate([src, ipad]).reshape(NW, n_chunks, CHUNK)
    val3 = jnp.concatenate([vals, jnp.zeros((pad,), jnp.float32)]).reshape(
        NW, n_chunks, CHUNK)
    n_pad = -(-n_nodes // (8 * NW)) * 8 * NW

    partials = _gcn_sc_kernel(n_pad, d_feat, n_chunks)(
        dst3, src3, val3, embeds)
    return _partial_add(partials, n_nodes)


# speedup vs baseline: 1.1264x; 1.1264x over previous
import functools

import jax
import jax.numpy as jnp
from jax import lax
from jax.experimental import pallas as pl
from jax.experimental.pallas import tpu as pltpu
from jax.experimental.pallas import tpu_sc as plsc

NC = 2
NS = 16
NW = NC * NS
L = 16
CHUNK = 128
PC = 16


def _gcn_sc_kernel(n_pad, d_feat, n_chunks):
    mesh = plsc.VectorSubcoreMesh(core_axis_name="c", subcore_axis_name="s")
    rows_per_tile = n_pad // NS

    @functools.partial(
        pl.kernel,
        out_type=jax.ShapeDtypeStruct((NC, n_pad, d_feat), jnp.float32),
        mesh=mesh,
        scratch_types=[
            pltpu.VMEM((PC, CHUNK), jnp.int32),
            pltpu.VMEM((PC, CHUNK), jnp.int32),
            pltpu.VMEM((PC, CHUNK), jnp.float32),
            pltpu.VMEM((CHUNK, d_feat), jnp.float32),
            pltpu.VMEM((CHUNK, d_feat), jnp.float32),
            pltpu.VMEM_SHARED((n_pad, d_feat), jnp.float32),
            pltpu.SemaphoreType.DMA,
            pltpu.SemaphoreType.DMA,
            pltpu.SemaphoreType.DMA,
            pltpu.SemaphoreType.DMA,
        ],
    )
    def k(dst_hbm, src_hbm, val_hbm, emb_hbm, out_hbm,
          dst_v, src_v, val_v, rows0, rows1, acc_sh, gs0, gs1, ss0, ss1):
        cid = lax.axis_index("c")
        sid = lax.axis_index("s")
        w = cid * NS + sid
        my_rows = pl.ds(sid * rows_per_tile, rows_per_tile)
        n_pass = n_chunks // PC

        zv = jnp.zeros((L,), jnp.float32)

        @pl.loop(0, CHUNK)
        def _(r):
            for j in range(d_feat // L):
                rows0[r, pl.ds(j * L, L)] = zv

        @pl.loop(0, rows_per_tile, step=CHUNK)
        def _(r0):
            pltpu.sync_copy(
                rows0, acc_sh.at[pl.ds(sid * rows_per_tile + r0, CHUNK)])

        plsc.subcore_barrier()

        def scale(buf, c):
            @pl.loop(0, CHUNK, step=L)
            def _(e0):
                vv = val_v[c, pl.ds(e0, L)]
                for i in range(L):
                    v = vv[i]
                    for j in range(d_feat // L):
                        sl = pl.ds(j * L, L)
                        buf[e0 + i, sl] = buf[e0 + i, sl] * v

        @pl.loop(0, n_pass)
        def _(h):
            hs = pl.ds(h * PC, PC)
            pltpu.sync_copy(dst_hbm.at[w, hs], dst_v)
            pltpu.sync_copy(src_hbm.at[w, hs], src_v)
            pltpu.sync_copy(val_hbm.at[w, hs], val_v)

            pltpu.async_copy(emb_hbm.at[src_v.at[0]], rows0, gs0)
            pltpu.async_copy(emb_hbm.at[src_v.at[1]], rows1, gs1)

            @pl.loop(0, PC, step=2)
            def _(c):
                pltpu.make_async_copy(emb_hbm.at[src_v.at[c]], rows0, gs0).wait()
                scale(rows0, c)
                pltpu.async_copy(rows0, acc_sh.at[dst_v.at[c]], ss0, add=True)

                pltpu.make_async_copy(emb_hbm.at[src_v.at[c + 1]], rows1, gs1).wait()
                scale(rows1, c + 1)
                pltpu.async_copy(rows1, acc_sh.at[dst_v.at[c + 1]], ss1, add=True)

                pltpu.make_async_copy(rows0, acc_sh.at[dst_v.at[c]], ss0).wait()

                @pl.when(c + 2 < PC)
                def _():
                    pltpu.async_copy(emb_hbm.at[src_v.at[c + 2]], rows0, gs0)

                pltpu.make_async_copy(rows1, acc_sh.at[dst_v.at[c + 1]], ss1).wait()

                @pl.when(c + 3 < PC)
                def _():
                    pltpu.async_copy(emb_hbm.at[src_v.at[c + 3]], rows1, gs1)

        plsc.subcore_barrier()
        pltpu.sync_copy(acc_sh.at[my_rows], out_hbm.at[cid, my_rows])

    return k


def _partial_add(partials, n_nodes):
    nc, n, d = partials.shape
    blk = 8 * NW * 8
    assert n % blk == 0

    def body(p_ref, o_ref):
        o_ref[...] = p_ref[0] + p_ref[1]

    return pl.pallas_call(
        body,
        out_shape=jax.ShapeDtypeStruct((n_nodes, d), jnp.float32),
        grid=(n // blk,),
        in_specs=[pl.BlockSpec((nc, blk, d), lambda i: (0, i, 0))],
        out_specs=pl.BlockSpec((blk, d), lambda i: (i, 0)),
    )(partials)


def kernel(edge_index, edge_vals, embeds):
    n_nodes, d_feat = embeds.shape
    n_edges = edge_vals.shape[0]
    n_chunks = -(--(-n_edges // (NW * CHUNK)) // PC) * PC
    per_tile = n_chunks * CHUNK
    pad = NW * per_tile - n_edges

    ipad = jnp.arange(pad, dtype=jnp.int32)
    if pad > n_nodes:
        ipad = ipad % jnp.int32(n_nodes)
    ei = jnp.concatenate(
        [edge_index.astype(jnp.int32),
         jnp.broadcast_to(ipad, (2, pad))], axis=1)
    dst3 = ei[0].reshape(NW, n_chunks, CHUNK)
    src3 = ei[1].reshape(NW, n_chunks, CHUNK)
    val3 = jnp.concatenate(
        [edge_vals.astype(jnp.float32), jnp.zeros((pad,), jnp.float32)]
    ).reshape(NW, n_chunks, CHUNK)
    n_pad = -(-n_nodes // (8 * NW)) * 8 * NW

    partials = _gcn_sc_kernel(n_pad, d_feat, n_chunks)(
        dst3, src3, val3, embeds)
    return _partial_add(partials, n_nodes)
